# Initial kernel scaffold; baseline (speedup 1.0000x reference)
import jax
import jax.numpy as jnp
from jax.experimental import pallas as pl


def kernel(data_in, times_in, mask_in, lengths, params):
    raise NotImplementedError("write your pallas kernel here")


# speedup vs baseline: 3.6616x; 3.6616x over previous
import functools
import numpy as np

import jax
import jax.numpy as jnp
from jax.experimental import pallas as pl

V = 36
D_OB = 4
T = 215
B = 128
VD = V * D_OB
D = T * D_OB
DIM_POS = 16
E_ENC = VD + DIM_POS
NHEAD = 8
DH = E_ENC // NHEAD
NHID = 512

f32 = jnp.float32
bf16 = jnp.bfloat16

_MM_NN = (((1,), (0,)), ((), ()))
_MM_NT = (((1,), (1,)), ((), ()))


def _mm(a, b, dims=_MM_NN):
    return jax.lax.dot_general(a.astype(bf16), b.astype(bf16),
                               dimension_numbers=dims,
                               preferred_element_type=f32)


def _split(x):
    hi = x.astype(bf16)
    lo = (x - hi.astype(f32)).astype(bf16)
    return hi, lo


def _mm3(a, b, dims=_MM_NN):
    ah, al = _split(a)
    bh, bl = _split(b)
    dot = functools.partial(jax.lax.dot_general, dimension_numbers=dims,
                            preferred_element_type=f32)
    return dot(ah, bh) + dot(ah, bl) + dot(al, bh)


_DIL_BT = 16


def _dil_kernel(di_ref, wh_ref, wl_ref, b_ref, out_ref):
    wh = wh_ref[...]
    wl = wl_ref[...]
    bias = b_ref[...]
    for i in range(_DIL_BT):
        db = di_ref[i]
        acc = jax.lax.dot_general(db, wh, dimension_numbers=_MM_NN,
                                  preferred_element_type=f32)
        acc = acc + jax.lax.dot_general(db, wl, dimension_numbers=_MM_NN,
                                        preferred_element_type=f32)
        out_ref[i] = jax.nn.relu(acc + bias)


_GAT_G = 8
_GAT_R = _GAT_G * V


def _gat_attn(q, k, v, mask, ew):
    logits = _mm3(q, k, dims=_MM_NT) / jnp.sqrt(f32(D))
    if ew is not None:
        logits = logits * ew
    logits = jnp.where(mask, logits, f32(-1e30))
    m = jnp.max(logits, axis=1, keepdims=True)
    e = jnp.exp(logits - m)
    s = jnp.sum(e, axis=1, keepdims=True)
    alpha = e / (s + 1e-16)
    out = _mm3(alpha, v, dims=_MM_NN)
    return out, alpha


def _gat_kernel(x_ref, wq1_ref, wk1_ref, wv1_ref, wq2_ref, wk2_ref, wv2_ref,
                y_ref, a2_ref):
    x = x_ref[...]
    row = jax.lax.broadcasted_iota(jnp.int32, (_GAT_R, _GAT_R), 0)
    col = jax.lax.broadcasted_iota(jnp.int32, (_GAT_R, _GAT_R), 1)
    mask = (row // V) == (col // V)

    q1 = jax.lax.dot_general(x, wq1_ref[...], dimension_numbers=_MM_NN,
                             preferred_element_type=f32)
    k1 = jax.lax.dot_general(x, wk1_ref[...], dimension_numbers=_MM_NN,
                             preferred_element_type=f32)
    v1 = jax.lax.dot_general(x, wv1_ref[...], dimension_numbers=_MM_NN,
                             preferred_element_type=f32)
    o1, a1 = _gat_attn(q1, k1, v1, mask, None)

    q2 = _mm(o1, wq2_ref[...])
    k2 = _mm(o1, wk2_ref[...])
    v2 = _mm(o1, wv2_ref[...])
    o2, a2 = _gat_attn(q2, k2, v2, mask, a1)

    y_ref[...] = o2
    for g in range(_GAT_G):
        a2_ref[g] = a2[V * g:V * (g + 1), V * g:V * (g + 1)]


_TR_BT = 8


def _ln(x, g, b):
    m = jnp.mean(x, axis=-1, keepdims=True)
    v = jnp.var(x, axis=-1, keepdims=True)
    return (x - m) / jnp.sqrt(v + 1e-5) * g + b


def _tr_kernel(g_ref, t_ref, ts_ref, pad_ref, *refs):
    lw, out_ref = refs[:-1], refs[-1]
    nl = 14

    def body(b, carry):
        tb = t_ref[b]
        scaled = tb[:, None] / ts_ref[...]
        pe = jnp.concatenate([jnp.sin(scaled), jnp.cos(scaled)], axis=-1)
        x = jnp.concatenate([g_ref[b], pe], axis=-1)
        padb = pad_ref[b][None, :] > 0.5
        for l in range(2):
            (wq, bq, wk, bk, wv, bv, wo, bo,
             w1, b1, w2, b2, ln1, ln2) = lw[l * nl:(l + 1) * nl]
            q = _mm(x, wq[...]) + bq[...]
            k = _mm(x, wk[...]) + bk[...]
            v = _mm(x, wv[...]) + bv[...]
            ohs = []
            for h in range(NHEAD):
                sl = slice(DH * h, DH * (h + 1))
                qh, kh, vh = q[:, sl], k[:, sl], v[:, sl]
                lg = _mm(qh, kh, dims=_MM_NT) / jnp.sqrt(f32(DH))
                lg = jnp.where(padb, f32(-1e9), lg)
                mx = jnp.max(lg, axis=1, keepdims=True)
                ee = jnp.exp(lg - mx)
                aa = ee / jnp.sum(ee, axis=1, keepdims=True)
                ohs.append(_mm(aa, vh))
            o = jnp.concatenate(ohs, axis=-1)
            attn = _mm(o, wo[...]) + bo[...]
            g1b1 = ln1[...]
            x = _ln(x + attn, g1b1[0:1, :], g1b1[1:2, :])
            ff = _mm(jax.nn.relu(_mm(x, w1[...]) + b1[...]), w2[...]) + b2[...]
            g2b2 = ln2[...]
            x = _ln(x + ff, g2b2[0:1, :], g2b2[1:2, :])
        out_ref[b] = x
        return carry

    jax.lax.fori_loop(0, _TR_BT, body, 0)


_DIST_BT = 8


def _dist_kernel(ai_ref, a_ref, o_ref):
    i = pl.program_id(0)
    ai = ai_ref[...]
    af = a_ref[...]
    d = ai[:, None, :] - af[None, :, :]
    s = jnp.sum(d * d, axis=-1)
    ds = jnp.sum(jnp.sqrt(s + 1e-12), keepdims=True).reshape(1, 1)
    prev = jnp.where(i == 0, jnp.zeros((1, 1), f32), o_ref[...])
    tot = prev + ds
    o_ref[...] = jnp.where(i == (B // _DIST_BT) - 1, tot / f32(B * B), tot)


def kernel(data_in, times_in, mask_in, lengths, params):
    p = params

    wdil = p["W_dil"].astype(bf16).astype(f32).reshape(V, D_OB, VD).sum(axis=1)
    wf_hi, wf_lo = _split(wdil)
    b_dil = p["b_dil"].reshape(1, VD)
    di = data_in.transpose(1, 0, 2).astype(bf16)

    h = pl.pallas_call(
        _dil_kernel,
        grid=(B // _DIL_BT,),
        in_specs=[
            pl.BlockSpec((_DIL_BT, T, V), lambda i: (i, 0, 0)),
            pl.BlockSpec((V, VD), lambda i: (0, 0)),
            pl.BlockSpec((V, VD), lambda i: (0, 0)),
            pl.BlockSpec((1, VD), lambda i: (0, 0)),
        ],
        out_specs=pl.BlockSpec((_DIL_BT, T, VD), lambda i: (i, 0, 0)),
        out_shape=jax.ShapeDtypeStruct((B, T, VD), f32),
    )(di, wf_hi, wf_lo, b_dil)

    xs = h.reshape(B, T, V, D_OB).transpose(0, 2, 1, 3).reshape(B * V, D)
    xs = xs.astype(bf16)

    gw = [p[n].astype(bf16) for n in ("Wq1", "Wk1", "Wv1", "Wq2", "Wk2", "Wv2")]
    y, a2 = pl.pallas_call(
        _gat_kernel,
        grid=(B // _GAT_G,),
        in_specs=[pl.BlockSpec((_GAT_R, D), lambda i: (i, 0))]
        + [pl.BlockSpec((D, D), lambda i: (0, 0))] * 6,
        out_specs=[
            pl.BlockSpec((_GAT_R, D), lambda i: (i, 0)),
            pl.BlockSpec((_GAT_G, V, V), lambda i: (i, 0, 0)),
        ],
        out_shape=[
            jax.ShapeDtypeStruct((B * V, D), f32),
            jax.ShapeDtypeStruct((B, V, V), f32),
        ],
    )(xs, *gw)

    gat_out = y.reshape(B, V, T, D_OB).transpose(0, 2, 1, 3).reshape(B, T, VD)

    times_t = times_in.T
    ts = jnp.asarray(
        np.power(float(T), np.linspace(0.0, 1.0, DIM_POS // 2))
        .astype(np.float32).reshape(1, DIM_POS // 2))
    pad = (jnp.arange(T)[None, :] >= lengths[:, None]).astype(f32)

    lw = []
    lw_specs = []
    for lp in p["layers"]:
        for wn, bn in (("Wq", "bq"), ("Wk", "bk"), ("Wv", "bv"), ("Wo", "bo")):
            lw += [lp[wn].astype(bf16), lp[bn].reshape(1, E_ENC)]
            lw_specs += [pl.BlockSpec((E_ENC, E_ENC), lambda i: (0, 0)),
                         pl.BlockSpec((1, E_ENC), lambda i: (0, 0))]
        lw += [lp["W1"].astype(bf16), lp["b1"].reshape(1, NHID),
               lp["W2"].astype(bf16), lp["b2"].reshape(1, E_ENC)]
        lw_specs += [pl.BlockSpec((E_ENC, NHID), lambda i: (0, 0)),
                     pl.BlockSpec((1, NHID), lambda i: (0, 0)),
                     pl.BlockSpec((NHID, E_ENC), lambda i: (0, 0)),
                     pl.BlockSpec((1, E_ENC), lambda i: (0, 0))]
        lw += [jnp.stack([lp["g1"], lp["be1"]]),
               jnp.stack([lp["g2"], lp["be2"]])]
        lw_specs += [pl.BlockSpec((2, E_ENC), lambda i: (0, 0))] * 2

    xout = pl.pallas_call(
        _tr_kernel,
        grid=(B // _TR_BT,),
        in_specs=[
            pl.BlockSpec((_TR_BT, T, VD), lambda i: (i, 0, 0)),
            pl.BlockSpec((_TR_BT, T), lambda i: (i, 0)),
            pl.BlockSpec((1, DIM_POS // 2), lambda i: (0, 0)),
            pl.BlockSpec((_TR_BT, T), lambda i: (i, 0)),
        ] + lw_specs,
        out_specs=pl.BlockSpec((_TR_BT, T, E_ENC), lambda i: (i, 0, 0)),
        out_shape=jax.ShapeDtypeStruct((B, T, E_ENC), f32),
    )(gat_out, times_t, ts, pad, *lw)

    x_final = xout.transpose(1, 0, 2)

    a2f = a2.reshape(B, V * V)
    dsum = pl.pallas_call(
        _dist_kernel,
        grid=(B // _DIST_BT,),
        in_specs=[
            pl.BlockSpec((_DIST_BT, V * V), lambda i: (i, 0)),
            pl.BlockSpec((B, V * V), lambda i: (0, 0)),
        ],
        out_specs=pl.BlockSpec((1, 1), lambda i: (0, 0)),
        out_shape=jax.ShapeDtypeStruct((1, 1), f32),
    )(a2f, a2f)
    distance = dsum[0, 0]

    return x_final, distance


# speedup vs baseline: 4.6663x; 1.2744x over previous
import functools
import numpy as np

import jax
import jax.numpy as jnp
from jax.experimental import pallas as pl

V = 36
D_OB = 4
T = 215
B = 128
VD = V * D_OB
D = T * D_OB
DIM_POS = 16
E_ENC = VD + DIM_POS
NHEAD = 8
DH = E_ENC // NHEAD
NHID = 512

f32 = jnp.float32
bf16 = jnp.bfloat16

_MM_NN = (((1,), (0,)), ((), ()))
_MM_NT = (((1,), (1,)), ((), ()))


def _mm(a, b, dims=_MM_NN):
    return jax.lax.dot_general(a.astype(bf16), b.astype(bf16),
                               dimension_numbers=dims,
                               preferred_element_type=f32)


def _split(x):
    hi = x.astype(bf16)
    lo = (x - hi.astype(f32)).astype(bf16)
    return hi, lo


def _mm3(a, b, dims=_MM_NN):
    ah, al = _split(a)
    bh, bl = _split(b)
    dot = functools.partial(jax.lax.dot_general, dimension_numbers=dims,
                            preferred_element_type=f32)
    return dot(ah, bh) + dot(ah, bl) + dot(al, bh)


_G = 8
_R = _G * V


def _gat_attn(q, k, v, mask, ew):
    logits = _mm3(q, k, dims=_MM_NT) / jnp.sqrt(f32(D))
    if ew is not None:
        logits = logits * ew
    logits = jnp.where(mask, logits, f32(-1e30))
    m = jnp.max(logits, axis=1, keepdims=True)
    e = jnp.exp(logits - m)
    s = jnp.sum(e, axis=1, keepdims=True)
    alpha = e / (s + 1e-16)
    out = _mm3(alpha, v, dims=_MM_NN)
    return out, alpha


def _ln(x, g, b):
    m = jnp.mean(x, axis=-1, keepdims=True)
    v = jnp.var(x, axis=-1, keepdims=True)
    return (x - m) / jnp.sqrt(v + 1e-5) * g + b


def _mega_kernel(data_ref, times_ref, ts_ref, pad_ref,
                 wfh_ref, wfl_ref, bdil_ref,
                 wq1_ref, wk1_ref, wv1_ref, wq2_ref, wk2_ref, wv2_ref,
                 p_ref, *refs):
    lw = refs[:-2]
    out_ref, a2_ref = refs[-2], refs[-1]
    nl = 14

    wfh = wfh_ref[...]
    wfl = wfl_ref[...]
    bdil = bdil_ref[...]
    cols = []
    for s in range(_G):
        d_s = data_ref[:, s, :]
        hp = jax.lax.dot_general(d_s, wfh, dimension_numbers=_MM_NN,
                                 preferred_element_type=f32)
        hp = hp + jax.lax.dot_general(d_s, wfl, dimension_numbers=_MM_NN,
                                      preferred_element_type=f32)
        hp = jax.nn.relu(hp + bdil)
        xt_s = jnp.concatenate([hp[:, V * j:V * (j + 1)] for j in range(4)],
                               axis=0)
        cols.append(xt_s)
    xt = jnp.concatenate(cols, axis=1)
    x = jnp.transpose(xt)

    row = jax.lax.broadcasted_iota(jnp.int32, (_R, _R), 0)
    col = jax.lax.broadcasted_iota(jnp.int32, (_R, _R), 1)
    mask = (row // V) == (col // V)

    xb = x.astype(bf16)
    q1 = jax.lax.dot_general(xb, wq1_ref[...], dimension_numbers=_MM_NN,
                             preferred_element_type=f32)
    k1 = jax.lax.dot_general(xb, wk1_ref[...], dimension_numbers=_MM_NN,
                             preferred_element_type=f32)
    v1 = jax.lax.dot_general(xb, wv1_ref[...], dimension_numbers=_MM_NN,
                             preferred_element_type=f32)
    o1, a1 = _gat_attn(q1, k1, v1, mask, None)

    q2 = _mm(o1, wq2_ref[...])
    k2 = _mm(o1, wk2_ref[...])
    v2 = _mm(o1, wv2_ref[...])
    o2, a2 = _gat_attn(q2, k2, v2, mask, a1)

    for s in range(_G):
        a2_ref[s] = a2[V * s:V * (s + 1), V * s:V * (s + 1)]

    o2t = jnp.transpose(o2)

    ts = ts_ref[...]
    pmat = p_ref[...]
    for s in range(_G):
        x144 = jnp.concatenate(
            [o2t[T * j:T * (j + 1), V * s:V * (s + 1)] for j in range(4)],
            axis=1)
        tb = times_ref[s]
        scaled = tb[:, None] / ts
        pe = jnp.concatenate([jnp.sin(scaled), jnp.cos(scaled)], axis=-1)
        xs = jnp.concatenate([x144, pe], axis=-1)
        padb = pad_ref[s][None, :] > 0.5
        for l in range(2):
            (wq, bq, wk, bk, wv, bv, wo, bo,
             w1, b1, w2, b2, ln1, ln2) = lw[l * nl:(l + 1) * nl]
            q = _mm(xs, wq[...]) + bq[...]
            k = _mm(xs, wk[...]) + bk[...]
            v = _mm(xs, wv[...]) + bv[...]
            ohs = []
            for h in range(NHEAD):
                sl = slice(DH * h, DH * (h + 1))
                qh, kh, vh = q[:, sl], k[:, sl], v[:, sl]
                lg = _mm(qh, kh, dims=_MM_NT) / jnp.sqrt(f32(DH))
                lg = jnp.where(padb, f32(-1e9), lg)
                mx = jnp.max(lg, axis=1, keepdims=True)
                ee = jnp.exp(lg - mx)
                aa = ee / jnp.sum(ee, axis=1, keepdims=True)
                ohs.append(_mm(aa, vh))
            o = jnp.concatenate(ohs, axis=-1)
            attn = _mm(o, wo[...]) + bo[...]
            g1b1 = ln1[...]
            xs = _ln(xs + attn, g1b1[0:1, :], g1b1[1:2, :])
            ff = _mm(jax.nn.relu(_mm(xs, w1[...]) + b1[...]), w2[...]) + b2[...]
            g2b2 = ln2[...]
            xs = _ln(xs + ff, g2b2[0:1, :], g2b2[1:2, :])
        xh, xl = _split(xs)
        xfin = (jax.lax.dot_general(xh, pmat, dimension_numbers=_MM_NN,
                                    preferred_element_type=f32)
                + jax.lax.dot_general(xl, pmat, dimension_numbers=_MM_NN,
                                      preferred_element_type=f32))
        out_ref[:, s, :] = xfin


_DIST_BT = 8


def _dist_kernel(ai_ref, a_ref, o_ref):
    i = pl.program_id(0)
    ai = ai_ref[...]
    af = a_ref[...]
    d = ai[:, None, :] - af[None, :, :]
    s = jnp.sum(d * d, axis=-1)
    ds = jnp.sum(jnp.sqrt(s + 1e-12), keepdims=True).reshape(1, 1)
    prev = jnp.where(i == 0, jnp.zeros((1, 1), f32), o_ref[...])
    tot = prev + ds
    o_ref[...] = jnp.where(i == (B // _DIST_BT) - 1, tot / f32(B * B), tot)


def _row_perm_d(w):
    return w.reshape(T, D_OB, D).transpose(1, 0, 2).reshape(D, D)


def _col_perm_d(w):
    return w.reshape(D, T, D_OB).transpose(0, 2, 1).reshape(D, D)


def _row_perm_e(w):
    top = w[:VD].reshape(V, D_OB, -1).transpose(1, 0, 2).reshape(VD, -1)
    return jnp.concatenate([top, w[VD:]], axis=0)


def _col_perm_e(w):
    top = w[:, :VD].reshape(-1, V, D_OB).transpose(0, 2, 1).reshape(w.shape[0], VD)
    return jnp.concatenate([top, w[:, VD:]], axis=1)


def _vec_perm_e(v):
    top = v[:VD].reshape(V, D_OB).T.reshape(VD)
    return jnp.concatenate([top, v[VD:]])


def kernel(data_in, times_in, mask_in, lengths, params):
    p = params

    wdil = p["W_dil"].astype(bf16).astype(f32).reshape(V, D_OB, VD).sum(axis=1)
    wfp = wdil.reshape(V, V, D_OB).transpose(0, 2, 1).reshape(V, VD)
    wf_hi, wf_lo = _split(wfp)
    bdil = p["b_dil"].reshape(V, D_OB).T.reshape(1, VD)

    wq1 = _row_perm_d(p["Wq1"]).astype(bf16)
    wk1 = _row_perm_d(p["Wk1"]).astype(bf16)
    wv1 = _row_perm_d(p["Wv1"]).astype(bf16)
    wq2 = p["Wq2"].astype(bf16)
    wk2 = p["Wk2"].astype(bf16)
    wv2 = _col_perm_d(p["Wv2"]).astype(bf16)

    pm = np.zeros((E_ENC, E_ENC), np.float32)
    for fp in range(VD):
        j, v = fp // V, fp % V
        pm[fp, 4 * v + j] = 1.0
    for fp in range(VD, E_ENC):
        pm[fp, fp] = 1.0
    pmat = jnp.asarray(pm).astype(bf16)

    lw = []
    lw_specs = []
    wspec = lambda r, c: pl.BlockSpec((r, c), lambda i: (0, 0))
    for lp in p["layers"]:
        for wn, bn in (("Wq", "bq"), ("Wk", "bk"), ("Wv", "bv")):
            lw += [_row_perm_e(lp[wn]).astype(bf16), lp[bn].reshape(1, E_ENC)]
            lw_specs += [wspec(E_ENC, E_ENC), wspec(1, E_ENC)]
        lw += [_col_perm_e(lp["Wo"]).astype(bf16),
               _vec_perm_e(lp["bo"]).reshape(1, E_ENC)]
        lw_specs += [wspec(E_ENC, E_ENC), wspec(1, E_ENC)]
        lw += [_row_perm_e(lp["W1"]).astype(bf16), lp["b1"].reshape(1, NHID),
               _col_perm_e(lp["W2"]).astype(bf16),
               _vec_perm_e(lp["b2"]).reshape(1, E_ENC)]
        lw_specs += [wspec(E_ENC, NHID), wspec(1, NHID),
                     wspec(NHID, E_ENC), wspec(1, E_ENC)]
        lw += [jnp.stack([_vec_perm_e(lp["g1"]), _vec_perm_e(lp["be1"])]),
               jnp.stack([_vec_perm_e(lp["g2"]), _vec_perm_e(lp["be2"])])]
        lw_specs += [wspec(2, E_ENC)] * 2

    data_bf = data_in.astype(bf16)
    times_t = times_in.T
    ts = jnp.asarray(
        np.power(float(T), np.linspace(0.0, 1.0, DIM_POS // 2))
        .astype(np.float32).reshape(1, DIM_POS // 2))
    pad = (jnp.arange(T)[None, :] >= lengths[:, None]).astype(f32)

    xout, a2 = pl.pallas_call(
        _mega_kernel,
        grid=(B // _G,),
        in_specs=[
            pl.BlockSpec((T, _G, V), lambda i: (0, i, 0)),
            pl.BlockSpec((_G, T), lambda i: (i, 0)),
            pl.BlockSpec((1, DIM_POS // 2), lambda i: (0, 0)),
            pl.BlockSpec((_G, T), lambda i: (i, 0)),
            wspec(V, VD), wspec(V, VD), wspec(1, VD),
            wspec(D, D), wspec(D, D), wspec(D, D),
            wspec(D, D), wspec(D, D), wspec(D, D),
            wspec(E_ENC, E_ENC),
        ] + lw_specs,
        out_specs=[
            pl.BlockSpec((T, _G, E_ENC), lambda i: (0, i, 0)),
            pl.BlockSpec((_G, V, V), lambda i: (i, 0, 0)),
        ],
        out_shape=[
            jax.ShapeDtypeStruct((T, B, E_ENC), f32),
            jax.ShapeDtypeStruct((B, V, V), f32),
        ],
    )(data_bf, times_t, ts, pad, wf_hi, wf_lo, bdil,
      wq1, wk1, wv1, wq2, wk2, wv2, pmat, *lw)

    a2f = a2.reshape(B, V * V)
    dsum = pl.pallas_call(
        _dist_kernel,
        grid=(B // _DIST_BT,),
        in_specs=[
            pl.BlockSpec((_DIST_BT, V * V), lambda i: (i, 0)),
            pl.BlockSpec((B, V * V), lambda i: (0, 0)),
        ],
        out_specs=pl.BlockSpec((1, 1), lambda i: (0, 0)),
        out_shape=jax.ShapeDtypeStruct((1, 1), f32),
    )(a2f, a2f)
    distance = dsum[0, 0]

    return xout, distance


# speedup vs baseline: 5.9466x; 1.2744x over previous
import functools
import numpy as np

import jax
import jax.numpy as jnp
from jax.experimental import pallas as pl

V = 36
D_OB = 4
T = 215
B = 128
VD = V * D_OB
D = T * D_OB
DIM_POS = 16
E_ENC = VD + DIM_POS
NHEAD = 8
DH = E_ENC // NHEAD
NHID = 512

f32 = jnp.float32
bf16 = jnp.bfloat16

_MM_NN = (((1,), (0,)), ((), ()))
_MM_NT = (((1,), (1,)), ((), ()))


def _mm(a, b, dims=_MM_NN):
    return jax.lax.dot_general(a.astype(bf16), b.astype(bf16),
                               dimension_numbers=dims,
                               preferred_element_type=f32)


def _split(x):
    hi = x.astype(bf16)
    lo = (x - hi.astype(f32)).astype(bf16)
    return hi, lo


def _mm3(a, b, dims=_MM_NN):
    ah, al = _split(a)
    bh, bl = _split(b)
    dot = functools.partial(jax.lax.dot_general, dimension_numbers=dims,
                            preferred_element_type=f32)
    return dot(ah, bh) + dot(ah, bl) + dot(al, bh)


_G = 8
_R = _G * V


def _gat_attn(q, k, v, mask, ew):
    logits = _mm3(q, k, dims=_MM_NT) / jnp.sqrt(f32(D))
    if ew is not None:
        logits = logits * ew
    logits = jnp.where(mask, logits, f32(-1e30))
    m = jnp.max(logits, axis=1, keepdims=True)
    e = jnp.exp(logits - m)
    s = jnp.sum(e, axis=1, keepdims=True)
    alpha = e / (s + 1e-16)
    out = _mm3(alpha, v, dims=_MM_NN)
    return out, alpha


def _ln(x, g, b):
    m = jnp.mean(x, axis=-1, keepdims=True)
    v = jnp.var(x, axis=-1, keepdims=True)
    return (x - m) / jnp.sqrt(v + 1e-5) * g + b


def _mega_kernel(data_ref, times_ref, ts_ref, pad_ref,
                 wfh_ref, wfl_ref, bdil_ref,
                 wq1_ref, wk1_ref, wv1_ref, wq2_ref, wk2_ref, wv2_ref,
                 p_ref, *refs):
    lw = refs[:-2]
    out_ref, a2_ref = refs[-2], refs[-1]
    nl = 14

    wfh = wfh_ref[...]
    wfl = wfl_ref[...]
    bdil = bdil_ref[...]
    cols = []
    for s in range(_G):
        d_s = data_ref[:, s, :]
        hp = jax.lax.dot_general(d_s, wfh, dimension_numbers=_MM_NN,
                                 preferred_element_type=f32)
        hp = hp + jax.lax.dot_general(d_s, wfl, dimension_numbers=_MM_NN,
                                      preferred_element_type=f32)
        hp = jax.nn.relu(hp + bdil)
        xt_s = jnp.concatenate([hp[:, V * j:V * (j + 1)] for j in range(4)],
                               axis=0)
        cols.append(xt_s)
    xt = jnp.concatenate(cols, axis=1)
    x = jnp.transpose(xt)

    row = jax.lax.broadcasted_iota(jnp.int32, (_R, _R), 0)
    col = jax.lax.broadcasted_iota(jnp.int32, (_R, _R), 1)
    mask = (row // V) == (col // V)

    xb = x.astype(bf16)
    q1 = jax.lax.dot_general(xb, wq1_ref[...], dimension_numbers=_MM_NN,
                             preferred_element_type=f32)
    k1 = jax.lax.dot_general(xb, wk1_ref[...], dimension_numbers=_MM_NN,
                             preferred_element_type=f32)
    v1 = jax.lax.dot_general(xb, wv1_ref[...], dimension_numbers=_MM_NN,
                             preferred_element_type=f32)
    o1, a1 = _gat_attn(q1, k1, v1, mask, None)

    q2 = _mm(o1, wq2_ref[...])
    k2 = _mm(o1, wk2_ref[...])
    v2 = _mm(o1, wv2_ref[...])
    o2, a2 = _gat_attn(q2, k2, v2, mask, a1)

    for s in range(_G):
        a2_ref[s] = a2[V * s:V * (s + 1), V * s:V * (s + 1)]

    o2t = jnp.transpose(o2)

    ts = ts_ref[...]
    pmat = p_ref[...]
    for s in range(_G):
        x144 = jnp.concatenate(
            [o2t[T * j:T * (j + 1), V * s:V * (s + 1)] for j in range(4)],
            axis=1)
        tb = times_ref[s]
        scaled = tb[:, None] / ts
        pe = jnp.concatenate([jnp.sin(scaled), jnp.cos(scaled)], axis=-1)
        xs = jnp.concatenate([x144, pe], axis=-1)
        padb = pad_ref[s][None, :] > 0.5
        for l in range(2):
            (wq, bq, wk, bk, wv, bv, wo, bo,
             w1, b1, w2, b2, ln1, ln2) = lw[l * nl:(l + 1) * nl]
            q = _mm(xs, wq[...]) + bq[...]
            k = _mm(xs, wk[...]) + bk[...]
            v = _mm(xs, wv[...]) + bv[...]
            ohs = []
            for h in range(NHEAD):
                sl = slice(128 * h, 128 * (h + 1))
                qh, kh, vh = q[:, sl], k[:, sl], v[:, sl]
                lg = _mm(qh, kh, dims=_MM_NT) / jnp.sqrt(f32(DH))
                lg = jnp.where(padb, f32(-1e9), lg)
                mx = jnp.max(lg, axis=1, keepdims=True)
                ee = jnp.exp(lg - mx)
                aa = ee / jnp.sum(ee, axis=1, keepdims=True)
                ohs.append(_mm(aa, vh))
            o = jnp.concatenate(ohs, axis=-1)
            attn = _mm(o, wo[...]) + bo[...]
            g1b1 = ln1[...]
            xs = _ln(xs + attn, g1b1[0:1, :], g1b1[1:2, :])
            ff = _mm(jax.nn.relu(_mm(xs, w1[...]) + b1[...]), w2[...]) + b2[...]
            g2b2 = ln2[...]
            xs = _ln(xs + ff, g2b2[0:1, :], g2b2[1:2, :])
        xh, xl = _split(xs)
        xfin = (jax.lax.dot_general(xh, pmat, dimension_numbers=_MM_NN,
                                    preferred_element_type=f32)
                + jax.lax.dot_general(xl, pmat, dimension_numbers=_MM_NN,
                                      preferred_element_type=f32))
        out_ref[:, s, :] = xfin


_DIST_BT = 8


def _dist_kernel(ai_ref, a_ref, o_ref):
    i = pl.program_id(0)
    ai = ai_ref[...]
    af = a_ref[...]
    d = ai[:, None, :] - af[None, :, :]
    s = jnp.sum(d * d, axis=-1)
    ds = jnp.sum(jnp.sqrt(s + 1e-12), keepdims=True).reshape(1, 1)
    prev = jnp.where(i == 0, jnp.zeros((1, 1), f32), o_ref[...])
    tot = prev + ds
    o_ref[...] = jnp.where(i == (B // _DIST_BT) - 1, tot / f32(B * B), tot)


def _row_perm_d(w):
    return w.reshape(T, D_OB, D).transpose(1, 0, 2).reshape(D, D)


def _col_perm_d(w):
    return w.reshape(D, T, D_OB).transpose(0, 2, 1).reshape(D, D)


def _row_perm_e(w):
    top = w[:VD].reshape(V, D_OB, -1).transpose(1, 0, 2).reshape(VD, -1)
    return jnp.concatenate([top, w[VD:]], axis=0)


def _col_perm_e(w):
    top = w[:, :VD].reshape(-1, V, D_OB).transpose(0, 2, 1).reshape(w.shape[0], VD)
    return jnp.concatenate([top, w[:, VD:]], axis=1)


def _vec_perm_e(v):
    top = v[:VD].reshape(V, D_OB).T.reshape(VD)
    return jnp.concatenate([top, v[VD:]])


def kernel(data_in, times_in, mask_in, lengths, params):
    p = params

    wdil = p["W_dil"].astype(bf16).astype(f32).reshape(V, D_OB, VD).sum(axis=1)
    wfp = wdil.reshape(V, V, D_OB).transpose(0, 2, 1).reshape(V, VD)
    wf_hi, wf_lo = _split(wfp)
    bdil = p["b_dil"].reshape(V, D_OB).T.reshape(1, VD)

    wq1 = _row_perm_d(p["Wq1"]).astype(bf16)
    wk1 = _row_perm_d(p["Wk1"]).astype(bf16)
    wv1 = _row_perm_d(p["Wv1"]).astype(bf16)
    wq2 = p["Wq2"].astype(bf16)
    wk2 = p["Wk2"].astype(bf16)
    wv2 = _col_perm_d(p["Wv2"]).astype(bf16)

    pm = np.zeros((E_ENC, E_ENC), np.float32)
    for fp in range(VD):
        j, v = fp // V, fp % V
        pm[fp, 4 * v + j] = 1.0
    for fp in range(VD, E_ENC):
        pm[fp, fp] = 1.0
    pmat = jnp.asarray(pm).astype(bf16)

    hpad = NHEAD * 128
    def _ext_cols(w):
        out = jnp.zeros((w.shape[0], hpad), w.dtype)
        for h in range(NHEAD):
            out = out.at[:, 128 * h:128 * h + DH].set(
                w[:, DH * h:DH * (h + 1)])
        return out

    def _ext_rows(w):
        out = jnp.zeros((hpad, w.shape[1]), w.dtype)
        for h in range(NHEAD):
            out = out.at[128 * h:128 * h + DH, :].set(
                w[DH * h:DH * (h + 1), :])
        return out

    lw = []
    lw_specs = []
    wspec = lambda r, c: pl.BlockSpec((r, c), lambda i: (0, 0))
    for lp in p["layers"]:
        for wn, bn in (("Wq", "bq"), ("Wk", "bk"), ("Wv", "bv")):
            lw += [_ext_cols(_row_perm_e(lp[wn])).astype(bf16),
                   _ext_cols(lp[bn].reshape(1, E_ENC))]
            lw_specs += [wspec(E_ENC, hpad), wspec(1, hpad)]
        lw += [_ext_rows(_col_perm_e(lp["Wo"])).astype(bf16),
               _vec_perm_e(lp["bo"]).reshape(1, E_ENC)]
        lw_specs += [wspec(hpad, E_ENC), wspec(1, E_ENC)]
        lw += [_row_perm_e(lp["W1"]).astype(bf16), lp["b1"].reshape(1, NHID),
               _col_perm_e(lp["W2"]).astype(bf16),
               _vec_perm_e(lp["b2"]).reshape(1, E_ENC)]
        lw_specs += [wspec(E_ENC, NHID), wspec(1, NHID),
                     wspec(NHID, E_ENC), wspec(1, E_ENC)]
        lw += [jnp.stack([_vec_perm_e(lp["g1"]), _vec_perm_e(lp["be1"])]),
               jnp.stack([_vec_perm_e(lp["g2"]), _vec_perm_e(lp["be2"])])]
        lw_specs += [wspec(2, E_ENC)] * 2

    data_bf = data_in.astype(bf16)
    times_t = times_in.T
    ts = jnp.asarray(
        np.power(float(T), np.linspace(0.0, 1.0, DIM_POS // 2))
        .astype(np.float32).reshape(1, DIM_POS // 2))
    pad = (jnp.arange(T)[None, :] >= lengths[:, None]).astype(f32)

    xout, a2 = pl.pallas_call(
        _mega_kernel,
        grid=(B // _G,),
        in_specs=[
            pl.BlockSpec((T, _G, V), lambda i: (0, i, 0)),
            pl.BlockSpec((_G, T), lambda i: (i, 0)),
            pl.BlockSpec((1, DIM_POS // 2), lambda i: (0, 0)),
            pl.BlockSpec((_G, T), lambda i: (i, 0)),
            wspec(V, VD), wspec(V, VD), wspec(1, VD),
            wspec(D, D), wspec(D, D), wspec(D, D),
            wspec(D, D), wspec(D, D), wspec(D, D),
            wspec(E_ENC, E_ENC),
        ] + lw_specs,
        out_specs=[
            pl.BlockSpec((T, _G, E_ENC), lambda i: (0, i, 0)),
            pl.BlockSpec((_G, V, V), lambda i: (i, 0, 0)),
        ],
        out_shape=[
            jax.ShapeDtypeStruct((T, B, E_ENC), f32),
            jax.ShapeDtypeStruct((B, V, V), f32),
        ],
    )(data_bf, times_t, ts, pad, wf_hi, wf_lo, bdil,
      wq1, wk1, wv1, wq2, wk2, wv2, pmat, *lw)

    a2f = a2.reshape(B, V * V)
    dsum = pl.pallas_call(
        _dist_kernel,
        grid=(B // _DIST_BT,),
        in_specs=[
            pl.BlockSpec((_DIST_BT, V * V), lambda i: (i, 0)),
            pl.BlockSpec((B, V * V), lambda i: (0, 0)),
        ],
        out_specs=pl.BlockSpec((1, 1), lambda i: (0, 0)),
        out_shape=jax.ShapeDtypeStruct((1, 1), f32),
    )(a2f, a2f)
    distance = dsum[0, 0]

    return xout, distance


# speedup vs baseline: 6.7335x; 1.1323x over previous
import functools
import numpy as np

import jax
import jax.numpy as jnp
from jax.experimental import pallas as pl

V = 36
D_OB = 4
T = 215
B = 128
VD = V * D_OB
D = T * D_OB
DIM_POS = 16
E_ENC = VD + DIM_POS
NHEAD = 8
DH = E_ENC // NHEAD
NHID = 512

f32 = jnp.float32
bf16 = jnp.bfloat16

_MM_NN = (((1,), (0,)), ((), ()))
_MM_NT = (((1,), (1,)), ((), ()))


def _mm(a, b, dims=_MM_NN):
    return jax.lax.dot_general(a.astype(bf16), b.astype(bf16),
                               dimension_numbers=dims,
                               preferred_element_type=f32)


def _split(x):
    hi = x.astype(bf16)
    lo = (x - hi.astype(f32)).astype(bf16)
    return hi, lo


def _mm3(a, b, dims=_MM_NN):
    ah, al = _split(a)
    bh, bl = _split(b)
    dot = functools.partial(jax.lax.dot_general, dimension_numbers=dims,
                            preferred_element_type=f32)
    return dot(ah, bh) + dot(ah, bl) + dot(al, bh)


_G = 8
_R = _G * V


def _gat_attn(q, k, v, mask, ew):
    logits = _mm3(q, k, dims=_MM_NT) / jnp.sqrt(f32(D))
    if ew is not None:
        logits = logits * ew
    logits = jnp.where(mask, logits, f32(-1e30))
    m = jnp.max(logits, axis=1, keepdims=True)
    e = jnp.exp(logits - m)
    s = jnp.sum(e, axis=1, keepdims=True)
    alpha = e / (s + 1e-16)
    out = _mm3(alpha, v, dims=_MM_NN)
    return out, alpha


def _ln(x, g, b):
    m = jnp.mean(x, axis=-1, keepdims=True)
    v = jnp.var(x, axis=-1, keepdims=True)
    return (x - m) / jnp.sqrt(v + 1e-5) * g + b


def _mega_kernel(data_ref, times_ref, ts_ref, pad_ref,
                 wfh_ref, wfl_ref, bdil_ref,
                 wq1_ref, wk1_ref, wv1_ref, wq2_ref, wk2_ref, wv2_ref,
                 p_ref, *refs):
    lw = refs[:-2]
    out_ref, a2_ref = refs[-2], refs[-1]
    nl = 14

    wfh = wfh_ref[...]
    wfl = wfl_ref[...]
    bdil = bdil_ref[...]
    cols = []
    for s in range(_G):
        d_s = data_ref[:, s, :]
        hp = jax.lax.dot_general(d_s, wfh, dimension_numbers=_MM_NN,
                                 preferred_element_type=f32)
        hp = hp + jax.lax.dot_general(d_s, wfl, dimension_numbers=_MM_NN,
                                      preferred_element_type=f32)
        hp = jax.nn.relu(hp + bdil)
        xt_s = jnp.concatenate([hp[:, V * j:V * (j + 1)] for j in range(4)],
                               axis=0)
        cols.append(xt_s)
    xt = jnp.concatenate(cols, axis=1)
    x = jnp.transpose(xt)

    row = jax.lax.broadcasted_iota(jnp.int32, (_R, _R), 0)
    col = jax.lax.broadcasted_iota(jnp.int32, (_R, _R), 1)
    mask = (row // V) == (col // V)

    xb = x.astype(bf16)
    q1 = jax.lax.dot_general(xb, wq1_ref[...], dimension_numbers=_MM_NN,
                             preferred_element_type=f32)
    k1 = jax.lax.dot_general(xb, wk1_ref[...], dimension_numbers=_MM_NN,
                             preferred_element_type=f32)
    v1 = jax.lax.dot_general(xb, wv1_ref[...], dimension_numbers=_MM_NN,
                             preferred_element_type=f32)
    o1, a1 = _gat_attn(q1, k1, v1, mask, None)

    q2 = _mm(o1, wq2_ref[...])
    k2 = _mm(o1, wk2_ref[...])
    v2 = _mm(o1, wv2_ref[...])
    o2, a2 = _gat_attn(q2, k2, v2, mask, a1)

    for s in range(_G):
        a2_ref[s] = a2[V * s:V * (s + 1), V * s:V * (s + 1)]

    o2t = jnp.transpose(o2)

    ts = ts_ref[...]
    pmat = p_ref[...]
    TP = T + 1
    zrow = jnp.zeros((1, E_ENC), f32)
    pieces = []
    for s in range(_G):
        x144 = jnp.concatenate(
            [o2t[T * j:T * (j + 1), V * s:V * (s + 1)] for j in range(4)],
            axis=1)
        tb = times_ref[s]
        scaled = tb[:, None] / ts
        pe = jnp.concatenate([jnp.sin(scaled), jnp.cos(scaled)], axis=-1)
        xs = jnp.concatenate([x144, pe], axis=-1)
        pieces.append(jnp.concatenate([xs, zrow], axis=0))
    x = jnp.concatenate(pieces, axis=0)

    for l in range(2):
        (wq, bq, wk, bk, wv, bv, wo, bo,
         w1, b1, w2, b2, ln1, ln2) = lw[l * nl:(l + 1) * nl]
        q = _mm(x, wq[...]) + bq[...]
        k = _mm(x, wk[...]) + bk[...]
        v = _mm(x, wv[...]) + bv[...]
        o_pieces = []
        for s in range(_G):
            r = slice(TP * s, TP * (s + 1))
            padb = pad_ref[s][None, :] > 0.5
            ohs = []
            for h in range(NHEAD):
                cl = slice(128 * h, 128 * (h + 1))
                qh, kh, vh = q[r, cl], k[r, cl], v[r, cl]
                lg = _mm(qh, kh, dims=_MM_NT) / jnp.sqrt(f32(DH))
                lg = jnp.where(padb, f32(-1e9), lg)
                mx = jnp.max(lg, axis=1, keepdims=True)
                ee = jnp.exp(lg - mx)
                aa = ee / jnp.sum(ee, axis=1, keepdims=True)
                ohs.append(_mm(aa, vh))
            o_pieces.append(jnp.concatenate(ohs, axis=-1))
        o = jnp.concatenate(o_pieces, axis=0)
        attn = _mm(o, wo[...]) + bo[...]
        g1b1 = ln1[...]
        x = _ln(x + attn, g1b1[0:1, :], g1b1[1:2, :])
        ff = _mm(jax.nn.relu(_mm(x, w1[...]) + b1[...]), w2[...]) + b2[...]
        g2b2 = ln2[...]
        x = _ln(x + ff, g2b2[0:1, :], g2b2[1:2, :])

    xh, xl = _split(x)
    xfin = (jax.lax.dot_general(xh, pmat, dimension_numbers=_MM_NN,
                                preferred_element_type=f32)
            + jax.lax.dot_general(xl, pmat, dimension_numbers=_MM_NN,
                                  preferred_element_type=f32))
    for s in range(_G):
        out_ref[:, s, :] = xfin[TP * s:TP * s + T, :]


_DIST_BT = 8


def _dist_kernel(ai_ref, a_ref, o_ref):
    i = pl.program_id(0)
    ai = ai_ref[...]
    af = a_ref[...]
    d = ai[:, None, :] - af[None, :, :]
    s = jnp.sum(d * d, axis=-1)
    ds = jnp.sum(jnp.sqrt(s + 1e-12), keepdims=True).reshape(1, 1)
    prev = jnp.where(i == 0, jnp.zeros((1, 1), f32), o_ref[...])
    tot = prev + ds
    o_ref[...] = jnp.where(i == (B // _DIST_BT) - 1, tot / f32(B * B), tot)


def _row_perm_d(w):
    return w.reshape(T, D_OB, D).transpose(1, 0, 2).reshape(D, D)


def _col_perm_d(w):
    return w.reshape(D, T, D_OB).transpose(0, 2, 1).reshape(D, D)


def _row_perm_e(w):
    top = w[:VD].reshape(V, D_OB, -1).transpose(1, 0, 2).reshape(VD, -1)
    return jnp.concatenate([top, w[VD:]], axis=0)


def _col_perm_e(w):
    top = w[:, :VD].reshape(-1, V, D_OB).transpose(0, 2, 1).reshape(w.shape[0], VD)
    return jnp.concatenate([top, w[:, VD:]], axis=1)


def _vec_perm_e(v):
    top = v[:VD].reshape(V, D_OB).T.reshape(VD)
    return jnp.concatenate([top, v[VD:]])


def kernel(data_in, times_in, mask_in, lengths, params):
    p = params

    wdil = p["W_dil"].astype(bf16).astype(f32).reshape(V, D_OB, VD).sum(axis=1)
    wfp = wdil.reshape(V, V, D_OB).transpose(0, 2, 1).reshape(V, VD)
    wf_hi, wf_lo = _split(wfp)
    bdil = p["b_dil"].reshape(V, D_OB).T.reshape(1, VD)

    wq1 = _row_perm_d(p["Wq1"]).astype(bf16)
    wk1 = _row_perm_d(p["Wk1"]).astype(bf16)
    wv1 = _row_perm_d(p["Wv1"]).astype(bf16)
    wq2 = p["Wq2"].astype(bf16)
    wk2 = p["Wk2"].astype(bf16)
    wv2 = _col_perm_d(p["Wv2"]).astype(bf16)

    pm = np.zeros((E_ENC, E_ENC), np.float32)
    for fp in range(VD):
        j, v = fp // V, fp % V
        pm[fp, 4 * v + j] = 1.0
    for fp in range(VD, E_ENC):
        pm[fp, fp] = 1.0
    pmat = jnp.asarray(pm).astype(bf16)

    hpad = NHEAD * 128
    def _ext_cols(w):
        out = jnp.zeros((w.shape[0], hpad), w.dtype)
        for h in range(NHEAD):
            out = out.at[:, 128 * h:128 * h + DH].set(
                w[:, DH * h:DH * (h + 1)])
        return out

    def _ext_rows(w):
        out = jnp.zeros((hpad, w.shape[1]), w.dtype)
        for h in range(NHEAD):
            out = out.at[128 * h:128 * h + DH, :].set(
                w[DH * h:DH * (h + 1), :])
        return out

    lw = []
    lw_specs = []
    wspec = lambda r, c: pl.BlockSpec((r, c), lambda i: (0, 0))
    for lp in p["layers"]:
        for wn, bn in (("Wq", "bq"), ("Wk", "bk"), ("Wv", "bv")):
            lw += [_ext_cols(_row_perm_e(lp[wn])).astype(bf16),
                   _ext_cols(lp[bn].reshape(1, E_ENC))]
            lw_specs += [wspec(E_ENC, hpad), wspec(1, hpad)]
        lw += [_ext_rows(_col_perm_e(lp["Wo"])).astype(bf16),
               _vec_perm_e(lp["bo"]).reshape(1, E_ENC)]
        lw_specs += [wspec(hpad, E_ENC), wspec(1, E_ENC)]
        lw += [_row_perm_e(lp["W1"]).astype(bf16), lp["b1"].reshape(1, NHID),
               _col_perm_e(lp["W2"]).astype(bf16),
               _vec_perm_e(lp["b2"]).reshape(1, E_ENC)]
        lw_specs += [wspec(E_ENC, NHID), wspec(1, NHID),
                     wspec(NHID, E_ENC), wspec(1, E_ENC)]
        lw += [jnp.stack([_vec_perm_e(lp["g1"]), _vec_perm_e(lp["be1"])]),
               jnp.stack([_vec_perm_e(lp["g2"]), _vec_perm_e(lp["be2"])])]
        lw_specs += [wspec(2, E_ENC)] * 2

    data_bf = data_in.astype(bf16)
    times_t = times_in.T
    ts = jnp.asarray(
        np.power(float(T), np.linspace(0.0, 1.0, DIM_POS // 2))
        .astype(np.float32).reshape(1, DIM_POS // 2))
    pad = (jnp.arange(T)[None, :] >= lengths[:, None]).astype(f32)
    pad = jnp.concatenate([pad, jnp.ones((B, 1), f32)], axis=1)

    xout, a2 = pl.pallas_call(
        _mega_kernel,
        grid=(B // _G,),
        in_specs=[
            pl.BlockSpec((T, _G, V), lambda i: (0, i, 0)),
            pl.BlockSpec((_G, T), lambda i: (i, 0)),
            pl.BlockSpec((1, DIM_POS // 2), lambda i: (0, 0)),
            pl.BlockSpec((_G, T + 1), lambda i: (i, 0)),
            wspec(V, VD), wspec(V, VD), wspec(1, VD),
            wspec(D, D), wspec(D, D), wspec(D, D),
            wspec(D, D), wspec(D, D), wspec(D, D),
            wspec(E_ENC, E_ENC),
        ] + lw_specs,
        out_specs=[
            pl.BlockSpec((T, _G, E_ENC), lambda i: (0, i, 0)),
            pl.BlockSpec((_G, V, V), lambda i: (i, 0, 0)),
        ],
        out_shape=[
            jax.ShapeDtypeStruct((T, B, E_ENC), f32),
            jax.ShapeDtypeStruct((B, V, V), f32),
        ],
    )(data_bf, times_t, ts, pad, wf_hi, wf_lo, bdil,
      wq1, wk1, wv1, wq2, wk2, wv2, pmat, *lw)

    a2f = a2.reshape(B, V * V)
    dsum = pl.pallas_call(
        _dist_kernel,
        grid=(B // _DIST_BT,),
        in_specs=[
            pl.BlockSpec((_DIST_BT, V * V), lambda i: (i, 0)),
            pl.BlockSpec((B, V * V), lambda i: (0, 0)),
        ],
        out_specs=pl.BlockSpec((1, 1), lambda i: (0, 0)),
        out_shape=jax.ShapeDtypeStruct((1, 1), f32),
    )(a2f, a2f)
    distance = dsum[0, 0]

    return xout, distance
